# baseline (speedup 1.0000x reference)
import functools

import jax
import jax.numpy as jnp
from jax.experimental import pallas as pl
from jax.experimental.pallas import tpu as pltpu

_B, _D, _C, _K, _L = 64, 2048, 345, 40, 9
_CB0 = 24
_CBH = 15
_NBUF = 4
_LOOKAHEAD = 2


def _head_body(f_ref, WT_ref, brow_ref, bcol_ref, labr_ref, labc_ref,
               ptr_ref, logits_ref, nptr_ref, zT_ref, posv_ref):
    f = f_ref[...]
    WT = WT_ref[...]
    logits = jax.lax.dot_general(f, WT, (((1,), (1,)), ((), ())),
                                 preferred_element_type=jnp.float32)
    logits_ref[...] = logits + brow_ref[...]
    logitsT = jax.lax.dot_general(WT, f, (((1,), (1,)), ((), ())),
                                  preferred_element_type=jnp.float32)
    logitsT = logitsT + bcol_ref[...]
    normT = jnp.sqrt(jnp.sum(logitsT * logitsT, axis=0, keepdims=True))
    zT_ref[...] = logitsT / (normT + 1e-12)
    labr = labr_ref[...]
    labc = labc_ref[...]
    same = labc == labr
    rows = jax.lax.broadcasted_iota(jnp.int32, (_B, _B), 0)
    cols = jax.lax.broadcasted_iota(jnp.int32, (_B, _B), 1)
    occ = jnp.sum(jnp.where(same & (cols < rows), 1, 0), axis=1,
                  keepdims=True, dtype=jnp.int32)
    cids = jax.lax.broadcasted_iota(jnp.int32, (_B, _C), 1)
    onehot_lab = labc == cids
    ptr = ptr_ref[...]
    ptr_g = jnp.sum(jnp.where(onehot_lab, ptr, 0), axis=1,
                    keepdims=True, dtype=jnp.int32)
    posv_ref[...] = jax.lax.rem(ptr_g + occ, _K)
    counts = jnp.sum(jnp.where(onehot_lab, 1, 0), axis=0,
                     keepdims=True, dtype=jnp.int32)
    nptr_ref[...] = jax.lax.rem(ptr + counts, _K)


def _qz_body(lab_sm, pos_sm, qz_blk, zT_blk, out_blk):
    out_blk[...] = qz_blk[...]
    zTb = zT_blk[...]
    cio = jax.lax.broadcasted_iota(jnp.int32, (1, _C), 1)
    for i in range(_B):
        li = lab_sm[i]
        pi = pos_sm[i]
        old = out_blk[:, pi, :]
        out_blk[:, pi, :] = jnp.where(cio == li, zTb[:, i:i + 1], old)


def _qh_body(lab_sm, pos_sm, hT_hbm, qh_hbm, qh_out, bufs, sem_in, sem_out,
             sem_rows):
    nch = _C // _CBH

    def _in_cp(c):
        slot = c % _NBUF
        return pltpu.make_async_copy(qh_hbm.at[pl.ds(c * _CBH, _CBH)],
                                     bufs.at[slot], sem_in.at[slot])

    def _out_cp(c):
        slot = c % _NBUF
        return pltpu.make_async_copy(bufs.at[slot],
                                     qh_out.at[pl.ds(c * _CBH, _CBH)],
                                     sem_out.at[slot])

    in_cps, out_cps = {}, {}
    for c in range(min(_LOOKAHEAD, nch)):
        in_cps[c] = _in_cp(c)
        in_cps[c].start()
    for c in range(nch):
        nxt = c + _LOOKAHEAD
        if nxt < nch:
            if nxt >= _NBUF:
                out_cps[nxt - _NBUF].wait()
            in_cps[nxt] = _in_cp(nxt)
            in_cps[nxt].start()
        in_cps[c].wait()
        out_cps[c] = _out_cp(c)
        out_cps[c].start()
    for c in range(max(0, nch - _NBUF), nch):
        out_cps[c].wait()
    row_cps = []
    for i in range(_B):
        l = lab_sm[i]
        p = pos_sm[i]
        cp = pltpu.make_async_copy(hT_hbm.at[:, i, :],
                                   qh_out.at[l, :, p, :], sem_rows)
        cp.start()
        row_cps.append(cp)
    for cp in row_cps:
        cp.wait()


def kernel(f, labels, h, queue_z, queue_h, queue_ptr, W, b):
    B, D, C, K, L = _B, _D, _C, _K, _L
    labr = labels.reshape(1, B)
    labc = labels.reshape(B, 1)
    ptr2 = queue_ptr.reshape(1, C)
    brow = b.reshape(1, C)
    bcol = b.reshape(C, 1)
    WT = jnp.transpose(W)
    qzT = jnp.transpose(queue_z, (0, 2, 1))
    qhT = jnp.transpose(queue_h, (0, 2, 1, 3))
    hT = jnp.transpose(h, (1, 0, 2))

    vmem = functools.partial(pl.BlockSpec, memory_space=pltpu.MemorySpace.VMEM)
    hbm = functools.partial(pl.BlockSpec, memory_space=pl.ANY)

    logits, nptr, zT, posv = pl.pallas_call(
        _head_body,
        in_specs=[vmem()] * 7,
        out_specs=(vmem(),) * 4,
        out_shape=(
            jax.ShapeDtypeStruct((B, C), jnp.float32),
            jax.ShapeDtypeStruct((1, C), jnp.int32),
            jax.ShapeDtypeStruct((C, B), jnp.float32),
            jax.ShapeDtypeStruct((B, 1), jnp.int32),
        ),
    )(f, WT, brow, bcol, labr, labc, ptr2)

    nsteps = pl.cdiv(C, _CB0)
    new_qzT = pl.pallas_call(
        _qz_body,
        grid_spec=pltpu.PrefetchScalarGridSpec(
            num_scalar_prefetch=2,
            grid=(nsteps,),
            in_specs=[
                pl.BlockSpec((_CB0, K, C), lambda i, l, p: (i, 0, 0)),
                pl.BlockSpec((_CB0, B), lambda i, l, p: (i, 0)),
            ],
            out_specs=pl.BlockSpec((_CB0, K, C), lambda i, l, p: (i, 0, 0)),
        ),
        out_shape=jax.ShapeDtypeStruct((C, K, C), jnp.float32),
    )(labels, posv.reshape(B), qzT, zT)

    new_qhT = pl.pallas_call(
        _qh_body,
        grid_spec=pltpu.PrefetchScalarGridSpec(
            num_scalar_prefetch=2,
            grid=(1,),
            in_specs=[hbm(), hbm()],
            out_specs=hbm(),
            scratch_shapes=[
                pltpu.VMEM((_NBUF, _CBH, L, K, C), jnp.float32),
                pltpu.SemaphoreType.DMA((_NBUF,)),
                pltpu.SemaphoreType.DMA((_NBUF,)),
                pltpu.SemaphoreType.DMA,
            ],
        ),
        out_shape=jax.ShapeDtypeStruct((C, L, K, C), jnp.float32),
    )(labels, posv.reshape(B), hT, qhT)

    new_qz = jnp.transpose(new_qzT, (0, 2, 1))
    new_qh = jnp.transpose(new_qhT, (0, 2, 1, 3))
    return (logits, new_qz, new_qh, nptr.reshape(C))


# speedup vs baseline: 1.0275x; 1.0275x over previous
import functools

import jax
import jax.numpy as jnp
from jax.experimental import pallas as pl
from jax.experimental.pallas import tpu as pltpu

_B, _D, _C, _K, _L = 64, 2048, 345, 40, 9
_CB0 = 120
_CBH = 15
_NBUF = 4
_LOOKAHEAD = 2


def _head_body(f_ref, WT_ref, brow_ref, bcol_ref, labr_ref, labc_ref,
               ptr_ref, logits_ref, nptr_ref, zT_ref, posv_ref):
    f = f_ref[...]
    WT = WT_ref[...]
    logits = jax.lax.dot_general(f, WT, (((1,), (1,)), ((), ())),
                                 preferred_element_type=jnp.float32)
    logits_ref[...] = logits + brow_ref[...]
    logitsT = jax.lax.dot_general(WT, f, (((1,), (1,)), ((), ())),
                                  preferred_element_type=jnp.float32)
    logitsT = logitsT + bcol_ref[...]
    normT = jnp.sqrt(jnp.sum(logitsT * logitsT, axis=0, keepdims=True))
    zT_ref[...] = logitsT / (normT + 1e-12)
    labr = labr_ref[...]
    labc = labc_ref[...]
    same = labc == labr
    rows = jax.lax.broadcasted_iota(jnp.int32, (_B, _B), 0)
    cols = jax.lax.broadcasted_iota(jnp.int32, (_B, _B), 1)
    occ = jnp.sum(jnp.where(same & (cols < rows), 1, 0), axis=1,
                  keepdims=True, dtype=jnp.int32)
    cids = jax.lax.broadcasted_iota(jnp.int32, (_B, _C), 1)
    onehot_lab = labc == cids
    ptr = ptr_ref[...]
    ptr_g = jnp.sum(jnp.where(onehot_lab, ptr, 0), axis=1,
                    keepdims=True, dtype=jnp.int32)
    posv_ref[...] = jax.lax.rem(ptr_g + occ, _K)
    counts = jnp.sum(jnp.where(onehot_lab, 1, 0), axis=0,
                     keepdims=True, dtype=jnp.int32)
    nptr_ref[...] = jax.lax.rem(ptr + counts, _K)


def _qz_body(lab_sm, pos_sm, qz_blk, zT_blk, out_blk):
    out_blk[...] = qz_blk[...]
    zTb = zT_blk[...]
    cio = jax.lax.broadcasted_iota(jnp.int32, (1, _C), 1)
    for i in range(_B):
        li = lab_sm[i]
        pi = pos_sm[i]
        old = out_blk[:, pi, :]
        out_blk[:, pi, :] = jnp.where(cio == li, zTb[:, i:i + 1], old)


def _qh_body(lab_sm, pos_sm, hT_hbm, qh_hbm, qh_out, bufs, sem_in, sem_out,
             sem_rows):
    nch = _C // _CBH

    def _in_cp(c):
        slot = c % _NBUF
        return pltpu.make_async_copy(qh_hbm.at[pl.ds(c * _CBH, _CBH)],
                                     bufs.at[slot], sem_in.at[slot])

    def _out_cp(c):
        slot = c % _NBUF
        return pltpu.make_async_copy(bufs.at[slot],
                                     qh_out.at[pl.ds(c * _CBH, _CBH)],
                                     sem_out.at[slot])

    in_cps, out_cps = {}, {}
    for c in range(min(_LOOKAHEAD, nch)):
        in_cps[c] = _in_cp(c)
        in_cps[c].start()
    for c in range(nch):
        nxt = c + _LOOKAHEAD
        if nxt < nch:
            if nxt >= _NBUF:
                out_cps[nxt - _NBUF].wait()
            in_cps[nxt] = _in_cp(nxt)
            in_cps[nxt].start()
        in_cps[c].wait()
        out_cps[c] = _out_cp(c)
        out_cps[c].start()
    for c in range(max(0, nch - _NBUF), nch):
        out_cps[c].wait()
    row_cps = []
    for i in range(_B):
        l = lab_sm[i]
        p = pos_sm[i]
        cp = pltpu.make_async_copy(hT_hbm.at[:, i, :],
                                   qh_out.at[l, :, p, :], sem_rows)
        cp.start()
        row_cps.append(cp)
    for cp in row_cps:
        cp.wait()


def kernel(f, labels, h, queue_z, queue_h, queue_ptr, W, b):
    B, D, C, K, L = _B, _D, _C, _K, _L
    labr = labels.reshape(1, B)
    labc = labels.reshape(B, 1)
    ptr2 = queue_ptr.reshape(1, C)
    brow = b.reshape(1, C)
    bcol = b.reshape(C, 1)
    WT = jnp.transpose(W)
    qzT = jnp.transpose(queue_z, (0, 2, 1))
    qhT = jnp.transpose(queue_h, (0, 2, 1, 3))
    hT = jnp.transpose(h, (1, 0, 2))

    vmem = functools.partial(pl.BlockSpec, memory_space=pltpu.MemorySpace.VMEM)
    hbm = functools.partial(pl.BlockSpec, memory_space=pl.ANY)

    logits, nptr, zT, posv = pl.pallas_call(
        _head_body,
        in_specs=[vmem()] * 7,
        out_specs=(vmem(),) * 4,
        out_shape=(
            jax.ShapeDtypeStruct((B, C), jnp.float32),
            jax.ShapeDtypeStruct((1, C), jnp.int32),
            jax.ShapeDtypeStruct((C, B), jnp.float32),
            jax.ShapeDtypeStruct((B, 1), jnp.int32),
        ),
    )(f, WT, brow, bcol, labr, labc, ptr2)

    nsteps = pl.cdiv(C, _CB0)
    new_qzT = pl.pallas_call(
        _qz_body,
        grid_spec=pltpu.PrefetchScalarGridSpec(
            num_scalar_prefetch=2,
            grid=(nsteps,),
            in_specs=[
                pl.BlockSpec((_CB0, K, C), lambda i, l, p: (i, 0, 0)),
                pl.BlockSpec((_CB0, B), lambda i, l, p: (i, 0)),
            ],
            out_specs=pl.BlockSpec((_CB0, K, C), lambda i, l, p: (i, 0, 0)),
        ),
        out_shape=jax.ShapeDtypeStruct((C, K, C), jnp.float32),
    )(labels, posv.reshape(B), qzT, zT)

    new_qhT = pl.pallas_call(
        _qh_body,
        grid_spec=pltpu.PrefetchScalarGridSpec(
            num_scalar_prefetch=2,
            grid=(1,),
            in_specs=[hbm(), hbm()],
            out_specs=hbm(),
            scratch_shapes=[
                pltpu.VMEM((_NBUF, _CBH, L, K, C), jnp.float32),
                pltpu.SemaphoreType.DMA((_NBUF,)),
                pltpu.SemaphoreType.DMA((_NBUF,)),
                pltpu.SemaphoreType.DMA,
            ],
        ),
        out_shape=jax.ShapeDtypeStruct((C, L, K, C), jnp.float32),
    )(labels, posv.reshape(B), hT, qhT)

    new_qz = jnp.transpose(new_qzT, (0, 2, 1))
    new_qh = jnp.transpose(new_qhT, (0, 2, 1, 3))
    return (logits, new_qz, new_qh, nptr.reshape(C))


# speedup vs baseline: 1.0301x; 1.0025x over previous
import functools

import jax
import jax.numpy as jnp
from jax.experimental import pallas as pl
from jax.experimental.pallas import tpu as pltpu

_B, _D, _C, _K, _L = 64, 2048, 345, 40, 9
_CB0 = 120
_CBH = 23
_NBUF = 3
_LOOKAHEAD = 2


def _head_body(f_ref, WT_ref, brow_ref, bcol_ref, labr_ref, labc_ref,
               ptr_ref, logits_ref, nptr_ref, zT_ref, posv_ref):
    f = f_ref[...]
    WT = WT_ref[...]
    logits = jax.lax.dot_general(f, WT, (((1,), (1,)), ((), ())),
                                 preferred_element_type=jnp.float32)
    logits_ref[...] = logits + brow_ref[...]
    logitsT = jax.lax.dot_general(WT, f, (((1,), (1,)), ((), ())),
                                  preferred_element_type=jnp.float32)
    logitsT = logitsT + bcol_ref[...]
    normT = jnp.sqrt(jnp.sum(logitsT * logitsT, axis=0, keepdims=True))
    zT_ref[...] = logitsT / (normT + 1e-12)
    labr = labr_ref[...]
    labc = labc_ref[...]
    same = labc == labr
    rows = jax.lax.broadcasted_iota(jnp.int32, (_B, _B), 0)
    cols = jax.lax.broadcasted_iota(jnp.int32, (_B, _B), 1)
    occ = jnp.sum(jnp.where(same & (cols < rows), 1, 0), axis=1,
                  keepdims=True, dtype=jnp.int32)
    cids = jax.lax.broadcasted_iota(jnp.int32, (_B, _C), 1)
    onehot_lab = labc == cids
    ptr = ptr_ref[...]
    ptr_g = jnp.sum(jnp.where(onehot_lab, ptr, 0), axis=1,
                    keepdims=True, dtype=jnp.int32)
    posv_ref[...] = jax.lax.rem(ptr_g + occ, _K)
    counts = jnp.sum(jnp.where(onehot_lab, 1, 0), axis=0,
                     keepdims=True, dtype=jnp.int32)
    nptr_ref[...] = jax.lax.rem(ptr + counts, _K)


def _qz_body(lab_sm, pos_sm, qz_blk, zT_blk, out_blk):
    out_blk[...] = qz_blk[...]
    zTb = zT_blk[...]
    cio = jax.lax.broadcasted_iota(jnp.int32, (1, _C), 1)
    for i in range(_B):
        li = lab_sm[i]
        pi = pos_sm[i]
        old = out_blk[:, pi, :]
        out_blk[:, pi, :] = jnp.where(cio == li, zTb[:, i:i + 1], old)


def _qh_body(lab_sm, pos_sm, hT_hbm, qh_hbm, qh_out, bufs, sem_in, sem_out,
             sem_rows):
    nch = _C // _CBH

    def _in_cp(c):
        slot = c % _NBUF
        return pltpu.make_async_copy(qh_hbm.at[pl.ds(c * _CBH, _CBH)],
                                     bufs.at[slot], sem_in.at[slot])

    def _out_cp(c):
        slot = c % _NBUF
        return pltpu.make_async_copy(bufs.at[slot],
                                     qh_out.at[pl.ds(c * _CBH, _CBH)],
                                     sem_out.at[slot])

    in_cps, out_cps = {}, {}
    for c in range(min(_LOOKAHEAD, nch)):
        in_cps[c] = _in_cp(c)
        in_cps[c].start()
    for c in range(nch):
        nxt = c + _LOOKAHEAD
        if nxt < nch:
            if nxt >= _NBUF:
                out_cps[nxt - _NBUF].wait()
            in_cps[nxt] = _in_cp(nxt)
            in_cps[nxt].start()
        in_cps[c].wait()
        out_cps[c] = _out_cp(c)
        out_cps[c].start()
    for c in range(max(0, nch - _NBUF), nch):
        out_cps[c].wait()
    row_cps = []
    for i in range(_B):
        l = lab_sm[i]
        p = pos_sm[i]
        cp = pltpu.make_async_copy(hT_hbm.at[:, i, :],
                                   qh_out.at[l, :, p, :], sem_rows)
        cp.start()
        row_cps.append(cp)
    for cp in row_cps:
        cp.wait()


def kernel(f, labels, h, queue_z, queue_h, queue_ptr, W, b):
    B, D, C, K, L = _B, _D, _C, _K, _L
    labr = labels.reshape(1, B)
    labc = labels.reshape(B, 1)
    ptr2 = queue_ptr.reshape(1, C)
    brow = b.reshape(1, C)
    bcol = b.reshape(C, 1)
    WT = jnp.transpose(W)
    qzT = jnp.transpose(queue_z, (0, 2, 1))
    qhT = jnp.transpose(queue_h, (0, 2, 1, 3))
    hT = jnp.transpose(h, (1, 0, 2))

    vmem = functools.partial(pl.BlockSpec, memory_space=pltpu.MemorySpace.VMEM)
    hbm = functools.partial(pl.BlockSpec, memory_space=pl.ANY)

    logits, nptr, zT, posv = pl.pallas_call(
        _head_body,
        in_specs=[vmem()] * 7,
        out_specs=(vmem(),) * 4,
        out_shape=(
            jax.ShapeDtypeStruct((B, C), jnp.float32),
            jax.ShapeDtypeStruct((1, C), jnp.int32),
            jax.ShapeDtypeStruct((C, B), jnp.float32),
            jax.ShapeDtypeStruct((B, 1), jnp.int32),
        ),
    )(f, WT, brow, bcol, labr, labc, ptr2)

    nsteps = pl.cdiv(C, _CB0)
    new_qzT = pl.pallas_call(
        _qz_body,
        grid_spec=pltpu.PrefetchScalarGridSpec(
            num_scalar_prefetch=2,
            grid=(nsteps,),
            in_specs=[
                pl.BlockSpec((_CB0, K, C), lambda i, l, p: (i, 0, 0)),
                pl.BlockSpec((_CB0, B), lambda i, l, p: (i, 0)),
            ],
            out_specs=pl.BlockSpec((_CB0, K, C), lambda i, l, p: (i, 0, 0)),
        ),
        out_shape=jax.ShapeDtypeStruct((C, K, C), jnp.float32),
    )(labels, posv.reshape(B), qzT, zT)

    new_qhT = pl.pallas_call(
        _qh_body,
        grid_spec=pltpu.PrefetchScalarGridSpec(
            num_scalar_prefetch=2,
            grid=(1,),
            in_specs=[hbm(), hbm()],
            out_specs=hbm(),
            scratch_shapes=[
                pltpu.VMEM((_NBUF, _CBH, L, K, C), jnp.float32),
                pltpu.SemaphoreType.DMA((_NBUF,)),
                pltpu.SemaphoreType.DMA((_NBUF,)),
                pltpu.SemaphoreType.DMA,
            ],
        ),
        out_shape=jax.ShapeDtypeStruct((C, L, K, C), jnp.float32),
    )(labels, posv.reshape(B), hT, qhT)

    new_qz = jnp.transpose(new_qzT, (0, 2, 1))
    new_qh = jnp.transpose(new_qhT, (0, 2, 1, 3))
    return (logits, new_qz, new_qh, nptr.reshape(C))
